# baseline (speedup 1.0000x reference)
import functools

import jax
import jax.numpy as jnp
from jax import lax
from jax.experimental import pallas as pl
from jax.experimental.pallas import tpu as pltpu
from jax.experimental.pallas import tpu_sc as plsc

_B, _S, _J, _H, _W, _P = 2, 2, 17, 128, 128, 8
_HW = _H * _W
_C = (1 + 1) * _J
_NJ = 32


def _mt_body(preds_hbm, kp_hbm, out_hbm, kp_v, val_v, sem):
    cid = lax.axis_index("c")
    sid = lax.axis_index("s")
    wid = sid * 2 + cid
    bs = wid // _P
    p = wid % _P
    b = bs // _S
    pltpu.sync_copy(kp_hbm.at[b, p], kp_v)
    base = bs * (_C * _HW) + _J * _HW
    acc = jnp.zeros((16,), jnp.float32)
    cnt = jnp.zeros((16,), jnp.float32)
    for c in range(_NJ // 16):
        k0 = kp_v[16 * c:16 * (c + 1)]
        k1 = kp_v[_NJ + 16 * c:_NJ + 16 * (c + 1)]
        idx = k0 + base
        pltpu.async_copy(preds_hbm.at[idx], val_v, sem).wait()
        vis = jnp.where(k1 > 0, 1.0, 0.0).astype(jnp.float32)
        acc = acc + val_v[...] * vis
        cnt = cnt + vis
    num = 0.0
    den = 0.0
    for i in range(16):
        num += acc[i]
        den += cnt[i]
    val_v[...] = jnp.broadcast_to(num, (16,)) / jnp.maximum(
        jnp.broadcast_to(den, (16,)), 1.0)
    pltpu.sync_copy(val_v, out_hbm.at[wid])


@functools.cache
def _mt_kernel():
    return pl.kernel(
        _mt_body,
        mesh=plsc.VectorSubcoreMesh(core_axis_name="c", subcore_axis_name="s"),
        out_type=jax.ShapeDtypeStruct((32, 16), jnp.float32),
        scratch_types=[
            pltpu.VMEM((2 * _NJ,), jnp.int32),
            pltpu.VMEM((16,), jnp.float32),
            pltpu.SemaphoreType.DMA,
        ],
    )


def _mean_tags(preds, gt_keypoints):
    kp = gt_keypoints.astype(jnp.int32)
    pad = ((0, 0), (0, 0), (0, _NJ - _J))
    idx = jnp.pad(kp[..., 0], pad)
    vis = jnp.pad(kp[..., 1], pad)
    packed = jnp.concatenate([idx, vis], axis=-1)
    return _mt_kernel()(preds.reshape(-1), packed)


def _loss_body(mt_ref, skel_ref, hm_ref, tg_ref, out_ref, acc_ref):
    jb = pl.program_id(0)

    @pl.when(jb == 0)
    def _():
        acc_ref[...] = jnp.zeros((2 * _B, _P), jnp.float32)

    si = lax.broadcasted_iota(jnp.int32, (_P, _H), 0)
    li = lax.broadcasted_iota(jnp.int32, (_P, _H), 1).astype(jnp.float32)
    y2 = jnp.where(si == 0, 1.0, jnp.where(si == 1, li, 0.0))
    xsr = lax.broadcasted_iota(jnp.int32, (_P, _W), 1).astype(jnp.float32)
    ones = jnp.ones((_W, _P), jnp.float32)
    g = skel_ref[0]
    units = [(b, s) for b in range(_B) for s in range(_S)]
    cmms = []
    for u, (b, s) in enumerate(units):
        tg = tg_ref[b, s, 0]
        base = 30.0 * hm_ref[b, s, 0] - 60.0 * (tg * tg)
        cms = []
        for p in range(_P):
            logit = base + (120.0 * mt_ref[u * _P + p, 0]) * tg
            cms.append(jnp.max(logit, axis=0, keepdims=True))
        cmms.append(jnp.concatenate(cms, axis=0))
    fs = []
    fxs = []
    for u in range(len(units)):
        mx8 = jnp.max(lax.transpose(cmms[u], (1, 0)), axis=0, keepdims=True)
        mxc = lax.transpose(mx8, (1, 0))
        f = jnp.exp(cmms[u] - mxc)
        fs.append(f)
        fxs.append(f * xsr)
    s3s = []
    for u, (b, s) in enumerate(units):
        cmm, f, fx = cmms[u], fs[u], fxs[u]
        tg = tg_ref[b, s, 0]
        base = 30.0 * hm_ref[b, s, 0] - 60.0 * (tg * tg)
        zrows = []
        syrows = []
        sxrows = []
        for p in range(_P):
            e = jnp.exp(base + (120.0 * mt_ref[u * _P + p, 0]) * tg
                        - cmm[p:p + 1, :])
            t = lax.dot_general(y2, e, (((1,), (0,)), ((), ())),
                                preferred_element_type=jnp.float32)
            zrows.append(t[0:1, :] * f[p:p + 1, :])
            syrows.append(t[1:2, :] * f[p:p + 1, :])
            sxrows.append(t[0:1, :] * fx[p:p + 1, :])
        s3s.append(jnp.concatenate(zrows + syrows + sxrows, axis=0))
    for u, (b, s) in enumerate(units):
        r = lax.dot_general(s3s[u], ones, (((1,), (0,)), ((), ())),
                            preferred_element_type=jnp.float32)
        rt = lax.transpose(r, (1, 0))
        zv = jnp.maximum(rt[0:1, 0:_P], 1e-30)
        syv = rt[0:1, _P:2 * _P]
        sxv = rt[0:1, 2 * _P:3 * _P]
        xv = sxv / zv
        yv = syv / zv
        gx = g[3 * b:3 * b + 1, 0:_P]
        gy = g[3 * b + 1:3 * b + 2, 0:_P]
        mk = jnp.where(g[3 * b + 2:3 * b + 3, 0:_P] > 0.0, 1.0, 0.0)
        dx = gx - xv
        dy = gy - yv
        contrib = mk * (dx * dx + dy * dy)
        acc_ref[2 * b:2 * b + 1, :] += contrib
        acc_ref[2 * b + 1:2 * b + 2, :] += mk

    @pl.when(jb == _J - 1)
    def _():
        tot = 0.0
        for bb in range(_B):
            num = jnp.sum(acc_ref[2 * bb:2 * bb + 1, :])
            den = jnp.sum(acc_ref[2 * bb + 1:2 * bb + 2, :])
            tot += num / jnp.maximum(den, 1.0)
        out_ref[0, 0] = tot / _B


def _loss_call(mt, skel_re, preds):
    return pl.pallas_call(
        _loss_body,
        grid=(_J,),
        in_specs=[
            pl.BlockSpec(memory_space=pltpu.SMEM),
            pl.BlockSpec((1, 8, _W), lambda j: (j, 0, 0)),
            pl.BlockSpec((_B, _S, 1, _H, _W), lambda j: (0, 0, j, 0, 0)),
            pl.BlockSpec((_B, _S, 1, _H, _W), lambda j: (0, 0, j + _J, 0, 0)),
        ],
        out_specs=pl.BlockSpec(memory_space=pltpu.SMEM),
        out_shape=jax.ShapeDtypeStruct((1, 1), jnp.float32),
        scratch_shapes=[
            pltpu.VMEM((2 * _B, _P), jnp.float32),
        ],
    )(mt, skel_re, preds, preds)


def _skel_rearrange(gt_skeletons):
    st = jnp.transpose(gt_skeletons, (2, 0, 3, 1))
    st = st.reshape(_J, _B * 3, _P)
    return jnp.pad(st, ((0, 0), (0, 8 - _B * 3), (0, _W - _P)))


def _mean_tags_jnp(preds, gt_keypoints):
    tags_flat = preds[:, :, _J:].reshape(_B, _S, _J * _HW)
    idx = gt_keypoints[..., 0].astype(jnp.int32)
    vis = (gt_keypoints[..., 1] > 0).astype(jnp.float32)
    gathered = jax.vmap(lambda tf, ix: tf[:, ix])(tags_flat, idx)
    cnt = jnp.clip(vis.sum(-1), 1.0)
    return (gathered * vis[:, None]).sum(-1) / cnt[:, None]


def kernel(preds, gt_masks, gt_skeletons, gt_heatmaps, gt_keypoints):
    mt = _mean_tags(preds, gt_keypoints)
    out = _loss_call(mt, _skel_rearrange(gt_skeletons), preds)
    return out[0, 0]


# speedup vs baseline: 1.0028x; 1.0028x over previous
import functools

import jax
import jax.numpy as jnp
from jax import lax
from jax.experimental import pallas as pl
from jax.experimental.pallas import tpu as pltpu
from jax.experimental.pallas import tpu_sc as plsc

_B, _S, _J, _H, _W, _P = 2, 2, 17, 128, 128, 8
_HW = _H * _W
_C = (1 + 1) * _J
_NJ = 32


def _mt_body(preds_hbm, kp_hbm, out_hbm, kp_v, val_v, sem):
    cid = lax.axis_index("c")
    sid = lax.axis_index("s")
    wid = sid * 2 + cid
    bs = wid // _P
    p = wid % _P
    b = bs // _S
    pltpu.sync_copy(kp_hbm.at[b, p], kp_v)
    base = bs * (_C * _HW) + _J * _HW
    acc = jnp.zeros((16,), jnp.float32)
    cnt = jnp.zeros((16,), jnp.float32)
    for c in range(_NJ // 16):
        k0 = kp_v[16 * c:16 * (c + 1)]
        k1 = kp_v[_NJ + 16 * c:_NJ + 16 * (c + 1)]
        idx = k0 + base
        pltpu.async_copy(preds_hbm.at[idx], val_v, sem).wait()
        vis = jnp.where(k1 > 0, 1.0, 0.0).astype(jnp.float32)
        acc = acc + val_v[...] * vis
        cnt = cnt + vis
    num = 0.0
    den = 0.0
    for i in range(16):
        num += acc[i]
        den += cnt[i]
    val_v[...] = jnp.broadcast_to(num, (16,)) / jnp.maximum(
        jnp.broadcast_to(den, (16,)), 1.0)
    pltpu.sync_copy(val_v, out_hbm.at[wid])


@functools.cache
def _mt_kernel():
    return pl.kernel(
        _mt_body,
        mesh=plsc.VectorSubcoreMesh(core_axis_name="c", subcore_axis_name="s"),
        out_type=jax.ShapeDtypeStruct((32, 16), jnp.float32),
        scratch_types=[
            pltpu.VMEM((2 * _NJ,), jnp.int32),
            pltpu.VMEM((16,), jnp.float32),
            pltpu.SemaphoreType.DMA,
        ],
    )


def _mean_tags(preds, gt_keypoints):
    kp = gt_keypoints.astype(jnp.int32)
    pad = ((0, 0), (0, 0), (0, _NJ - _J))
    idx = jnp.pad(kp[..., 0], pad)
    vis = jnp.pad(kp[..., 1], pad)
    packed = jnp.concatenate([idx, vis], axis=-1)
    return _mt_kernel()(preds.reshape(-1), packed)


def _loss_body(mt_ref, skel_ref, hm_ref, tg_ref, out_ref, acc_ref):
    jb = pl.program_id(0)

    @pl.when(jb == 0)
    def _():
        acc_ref[...] = jnp.zeros((2 * _B, _P), jnp.float32)

    si = lax.broadcasted_iota(jnp.int32, (_P, _H), 0)
    li = lax.broadcasted_iota(jnp.int32, (_P, _H), 1).astype(jnp.float32)
    y2 = jnp.where(si == 0, 1.0, jnp.where(si == 1, li, 0.0))
    xsr = lax.broadcasted_iota(jnp.int32, (_P, _W), 1).astype(jnp.float32)
    ones = jnp.ones((_W, _P), jnp.float32)
    g = skel_ref[0]
    units = [(b, s) for b in range(_B) for s in range(_S)]
    cmms = []
    for u, (b, s) in enumerate(units):
        tg = tg_ref[b, s, 0]
        base = 30.0 * hm_ref[b, s, 0] - 60.0 * (tg * tg)
        cms = []
        for p in range(_P):
            logit = base + (120.0 * mt_ref[u * _P + p, 0]) * tg
            cms.append(jnp.max(logit, axis=0, keepdims=True))
        cmms.append(jnp.concatenate(cms, axis=0))
    fs = []
    fxs = []
    for u in range(len(units)):
        mx8 = jnp.max(lax.transpose(cmms[u], (1, 0)), axis=0, keepdims=True)
        mxc = lax.transpose(mx8, (1, 0))
        f = jnp.exp(cmms[u] - mxc)
        fs.append(f)
        fxs.append(f * xsr)
    s3s = []
    for u, (b, s) in enumerate(units):
        cmm, f, fx = cmms[u], fs[u], fxs[u]
        tg = tg_ref[b, s, 0]
        base = 30.0 * hm_ref[b, s, 0] - 60.0 * (tg * tg)
        zrows = []
        syrows = []
        sxrows = []
        for p in range(_P):
            e = jnp.exp(base + (120.0 * mt_ref[u * _P + p, 0]) * tg
                        - cmm[p:p + 1, :])
            t = lax.dot_general(y2, e, (((1,), (0,)), ((), ())),
                                preferred_element_type=jnp.float32)
            zrows.append(t[0:1, :] * f[p:p + 1, :])
            syrows.append(t[1:2, :] * f[p:p + 1, :])
            sxrows.append(t[0:1, :] * fx[p:p + 1, :])
        s3s.append(jnp.concatenate(zrows + syrows + sxrows, axis=0))
    for u, (b, s) in enumerate(units):
        r = lax.dot_general(s3s[u], ones, (((1,), (0,)), ((), ())),
                            preferred_element_type=jnp.float32)
        rt = lax.transpose(r, (1, 0))
        zv = jnp.maximum(rt[0:1, 0:_P], 1e-30)
        syv = rt[0:1, _P:2 * _P]
        sxv = rt[0:1, 2 * _P:3 * _P]
        xv = sxv / zv
        yv = syv / zv
        gx = g[3 * b:3 * b + 1, 0:_P]
        gy = g[3 * b + 1:3 * b + 2, 0:_P]
        mk = jnp.where(g[3 * b + 2:3 * b + 3, 0:_P] > 0.0, 1.0, 0.0)
        dx = gx - xv
        dy = gy - yv
        contrib = mk * (dx * dx + dy * dy)
        acc_ref[2 * b:2 * b + 1, :] += contrib
        acc_ref[2 * b + 1:2 * b + 2, :] += mk

    @pl.when(jb == _J - 1)
    def _():
        tot = 0.0
        for bb in range(_B):
            num = jnp.sum(acc_ref[2 * bb:2 * bb + 1, :])
            den = jnp.sum(acc_ref[2 * bb + 1:2 * bb + 2, :])
            tot += num / jnp.maximum(den, 1.0)
        out_ref[0, 0] = tot / _B


def _loss_call(mt, skel_re, preds):
    return pl.pallas_call(
        _loss_body,
        grid=(_J,),
        in_specs=[
            pl.BlockSpec(memory_space=pltpu.SMEM),
            pl.BlockSpec((1, 8, _W), lambda j: (j, 0, 0)),
            pl.BlockSpec((_B, _S, 1, _H, _W), lambda j: (0, 0, j, 0, 0)),
            pl.BlockSpec((_B, _S, 1, _H, _W), lambda j: (0, 0, j + _J, 0, 0)),
        ],
        out_specs=pl.BlockSpec(memory_space=pltpu.SMEM),
        out_shape=jax.ShapeDtypeStruct((1, 1), jnp.float32),
        scratch_shapes=[
            pltpu.VMEM((2 * _B, _P), jnp.float32),
        ],
    )(mt, skel_re, preds, preds)


def _skel_rearrange(gt_skeletons):
    st = jnp.transpose(gt_skeletons, (2, 0, 3, 1))
    st = st.reshape(_J, _B * 3, _P)
    return jnp.pad(st, ((0, 0), (0, 8 - _B * 3), (0, _W - _P)))


def kernel(preds, gt_masks, gt_skeletons, gt_heatmaps, gt_keypoints):
    mt = _mean_tags(preds, gt_keypoints)
    out = _loss_call(mt, _skel_rearrange(gt_skeletons), preds)
    return out[0, 0]
